# baseline (speedup 1.0000x reference)
import functools

import jax
import jax.numpy as jnp
from jax import lax
from jax.experimental import pallas as pl
from jax.experimental.pallas import tpu as pltpu
from jax.experimental.pallas import tpu_sc as plsc

_NC = 2
_NS = 16
_NW = _NC * _NS
_L = 16
_K = 64


def _node_tables_body(x_ref, bt_ref, w1d_ref, w1s_ref, u_ref, w1u_ref,
                      b1_ref, a_ref, b_ref):
    n_graphs = u_ref.shape[0]
    cu = jnp.dot(u_ref[...], w1u_ref[...],
                 preferred_element_type=jnp.float32) + b1_ref[...]
    oh = (bt_ref[...] == lax.broadcasted_iota(jnp.int32, (1, n_graphs), 1)
          ).astype(jnp.float32)
    xb = x_ref[...]
    a_ref[...] = (jnp.dot(xb, w1d_ref[...], preferred_element_type=jnp.float32)
                  + jnp.dot(oh, cu, preferred_element_type=jnp.float32))
    b_ref[...] = jnp.dot(xb, w1s_ref[...], preferred_element_type=jnp.float32)


def _make_edge_term_body(n_edges, eblk):
    def _edge_term_body(ea_ref, w1e_ref, e_ref):
        i = pl.program_id(0)
        d = w1e_ref.shape[1]
        row = i * eblk + lax.broadcasted_iota(jnp.int32, (eblk, 1), 0)
        pad = jnp.where(row >= n_edges, jnp.float32(-1e30), jnp.float32(0.0))
        e_ref[...] = (jnp.dot(ea_ref[...], w1e_ref[...],
                              preferred_element_type=jnp.float32)
                      + pad).astype(jnp.bfloat16)
    return _edge_term_body


def _final_body(x_ref, agg_ref, bt_ref, w2x_ref, w2m_ref, u_ref, w2u_ref,
                b2_ref, o_ref):
    n_graphs = u_ref.shape[0]
    cu = jnp.dot(u_ref[...], w2u_ref[...],
                 preferred_element_type=jnp.float32) + b2_ref[...]
    oh = (bt_ref[...] == lax.broadcasted_iota(jnp.int32, (1, n_graphs), 1)
          ).astype(jnp.float32)
    agg = agg_ref[0] + agg_ref[1]
    acc = jnp.dot(x_ref[...], w2x_ref[...], preferred_element_type=jnp.float32)
    acc = acc + jnp.dot(agg, w2m_ref[...], preferred_element_type=jnp.float32)
    acc = acc + jnp.dot(oh, cu, preferred_element_type=jnp.float32)
    o_ref[...] = jnp.maximum(acc, 0.0)


def _make_sc_edge_kernel(n_nodes, n_chunks, d):
    assert n_chunks % _NW == 0
    n_iter = n_chunks // _NW
    assert n_iter % 2 == 0
    rpt = (n_nodes // _NS) // 8 * 8
    tail = n_nodes - _NS * rpt
    wchunks = [(o, _K) for o in range(0, rpt - rpt % _K, _K)]
    if rpt % _K:
        wchunks.append((rpt - rpt % _K, rpt % _K))
    nj = d // _L

    mesh = plsc.VectorSubcoreMesh(core_axis_name="c", subcore_axis_name="s",
                                  num_cores=_NC, num_subcores=_NS)

    @functools.partial(
        pl.kernel,
        out_type=jax.ShapeDtypeStruct((_NC, n_nodes, d), jnp.float32),
        mesh=mesh,
        scratch_types=(
            [pltpu.VMEM((2, _K), jnp.int32)] * 2
            + [pltpu.VMEM((_K, d), jnp.float32)] * 4
            + [pltpu.VMEM((_K, d // 2), jnp.int32)] * 2
            + [pltpu.VMEM_SHARED((n_nodes, d), jnp.float32)]
            + [pltpu.SemaphoreType.DMA] * 6
        ),
    )
    def sc_edge_kernel(a_hbm, b_hbm, e_hbm, didx_hbm, out_hbm, *scr):
        di = scr[0:2]
        ba = scr[2:4]
        bb = scr[4:6]
        be = scr[6:8]
        aggs = scr[8]
        sa = scr[9:11]
        sb = scr[11:13]
        se = scr[13:15]
        c = lax.axis_index("c")
        s = lax.axis_index("s")
        wid = s * _NC + c

        def gather_start(i, p):
            base = (i * _NW + wid) * _K
            pltpu.sync_copy(didx_hbm.at[i * _NW + wid], di[p])
            pltpu.make_async_copy(e_hbm.at[pl.ds(base, _K), :], be[p],
                                  se[p]).start()
            pltpu.make_async_copy(b_hbm.at[di[p].at[1]], bb[p], sb[p]).start()
            pltpu.make_async_copy(a_hbm.at[di[p].at[0]], ba[p], sa[p]).start()

        def gather_wait(i, p):
            base = (i * _NW + wid) * _K
            pltpu.make_async_copy(e_hbm.at[pl.ds(base, _K), :], be[p],
                                  se[p]).wait()
            pltpu.make_async_copy(b_hbm.at[di[p].at[1]], bb[p], sb[p]).wait()
            pltpu.make_async_copy(a_hbm.at[di[p].at[0]], ba[p], sa[p]).wait()

        gather_start(0, 0)

        def zero_row(r, carry):
            for j in range(nj):
                bb[1][r, pl.ds(j * _L, _L)] = jnp.zeros((_L,), jnp.float32)
            return carry
        lax.fori_loop(0, _K, zero_row, 0)
        for off, sz in wchunks:
            pltpu.sync_copy(bb[1].at[pl.ds(0, sz), :],
                            aggs.at[pl.ds(s * rpt + off, sz), :])
        if tail:
            @pl.when(s == 0)
            def _zero_tail():
                pltpu.sync_copy(bb[1].at[pl.ds(0, tail), :],
                                aggs.at[pl.ds(_NS * rpt, tail), :])
        plsc.subcore_barrier()

        def sub_iter(i, p):
            @pl.when(i + 1 < n_iter)
            def _():
                gather_start(i + 1, 1 - p)
            gather_wait(i, p)

            def msg_rows(r2, carry2):
                r = r2 * 2
                for rr in range(2):
                    for j in range(d // (2 * _L)):
                        ei = be[p][r + rr, pl.ds(j * _L, _L)]
                        elo = jax.lax.bitcast_convert_type(
                            ei << 16, jnp.float32)
                        ehi = jax.lax.bitcast_convert_type(
                            ei & jnp.int32(-65536), jnp.float32)
                        sl0 = pl.ds(j * 2 * _L, _L)
                        sl1 = pl.ds(j * 2 * _L + _L, _L)
                        v0 = ba[p][r + rr, sl0] + bb[p][r + rr, sl0] + elo
                        v1 = ba[p][r + rr, sl1] + bb[p][r + rr, sl1] + ehi
                        ba[p][r + rr, sl0] = jnp.maximum(v0, 0.0)
                        ba[p][r + rr, sl1] = jnp.maximum(v1, 0.0)
                return carry2
            lax.fori_loop(0, _K // 2, msg_rows, 0)

            pltpu.sync_copy(ba[p], aggs.at[di[p].at[0]], add=True)

        def loop_body(i2, carry):
            sub_iter(i2 * 2, 0)
            sub_iter(i2 * 2 + 1, 1)
            return carry
        lax.fori_loop(0, n_iter // 2, loop_body, 0)
        plsc.subcore_barrier()

        for off, sz in wchunks:
            r0 = s * rpt + off
            pltpu.sync_copy(aggs.at[pl.ds(r0, sz), :],
                            bb[1].at[pl.ds(0, sz), :])
            pltpu.sync_copy(bb[1].at[pl.ds(0, sz), :],
                            out_hbm.at[c, pl.ds(r0, sz), :])
        if tail:
            @pl.when(s == 0)
            def _write_tail():
                r0 = _NS * rpt
                pltpu.sync_copy(aggs.at[pl.ds(r0, tail), :],
                                bb[1].at[pl.ds(0, tail), :])
                pltpu.sync_copy(bb[1].at[pl.ds(0, tail), :],
                                out_hbm.at[c, pl.ds(r0, tail), :])

    return sc_edge_kernel


def kernel(x, edge_index, edge_attr, u, batch, W1, b1, W2, b2):
    n_nodes, d = x.shape
    n_edges, d_edge = edge_attr.shape
    n_graphs, d_u = u.shape
    assert n_nodes % _NS == 0

    grp = _NW * _K * 2
    n_pad = -(-n_edges // grp) * grp
    n_chunks = n_pad // _K

    dst = edge_index[1].astype(jnp.int32)
    src = edge_index[0].astype(jnp.int32)
    pad1 = jnp.zeros((n_pad - n_edges,), jnp.int32)
    didx = jnp.stack([
        jnp.concatenate([dst, pad1]).reshape(n_chunks, _K),
        jnp.concatenate([src, pad1]).reshape(n_chunks, _K)], axis=1)
    ea_pad = jnp.concatenate(
        [edge_attr, jnp.zeros((n_pad - n_edges, d_edge), jnp.float32)], axis=0)
    bt2 = batch.astype(jnp.int32).reshape(n_nodes, 1)

    q = []
    for j in range(d // 32):
        q.extend(32 * j + 2 * k for k in range(16))
        q.extend(32 * j + 2 * k + 1 for k in range(16))
    qarr = jnp.array(q, jnp.int32)
    W1d = W1[0:d][:, qarr]
    W1s = W1[d:2 * d][:, qarr]
    W1e = W1[2 * d:2 * d + d_edge]
    W1u = W1[2 * d + d_edge:][:, qarr]
    b1r = b1[qarr].reshape(1, d)
    W2x = W2[0:d]
    W2m = W2[d:2 * d][qarr]
    W2u = W2[2 * d:]
    b2r = b2.reshape(1, d)

    nblk = 1000
    ngrid = n_nodes // nblk
    a_tab, b_tab = pl.pallas_call(
        _node_tables_body,
        grid=(ngrid,),
        in_specs=[
            pl.BlockSpec((nblk, d), lambda i: (i, 0)),
            pl.BlockSpec((nblk, 1), lambda i: (i, 0)),
            pl.BlockSpec((d, d), lambda i: (0, 0)),
            pl.BlockSpec((d, d), lambda i: (0, 0)),
            pl.BlockSpec((n_graphs, d_u), lambda i: (0, 0)),
            pl.BlockSpec((d_u, d), lambda i: (0, 0)),
            pl.BlockSpec((1, d), lambda i: (0, 0)),
        ],
        out_specs=[
            pl.BlockSpec((nblk, d), lambda i: (i, 0)),
            pl.BlockSpec((nblk, d), lambda i: (i, 0)),
        ],
        out_shape=[
            jax.ShapeDtypeStruct((n_nodes, d), jnp.float32),
            jax.ShapeDtypeStruct((n_nodes, d), jnp.float32),
        ],
    )(x, bt2, W1d, W1s, u, W1u, b1r)

    eblk = 4096
    assert n_pad % eblk == 0
    egrid = n_pad // eblk
    e_term = pl.pallas_call(
        _make_edge_term_body(n_edges, eblk),
        grid=(egrid,),
        in_specs=[
            pl.BlockSpec((eblk, d_edge), lambda i: (i, 0)),
            pl.BlockSpec((d_edge, d), lambda i: (0, 0)),
        ],
        out_specs=pl.BlockSpec((eblk, d), lambda i: (i, 0)),
        out_shape=jax.ShapeDtypeStruct((n_pad, d), jnp.bfloat16),
    )(ea_pad, W1e)
    e_term = jax.lax.bitcast_convert_type(
        e_term.reshape(n_pad, d // 2, 2), jnp.int32)

    sc_edge = _make_sc_edge_kernel(n_nodes, n_chunks, d)
    agg_parts = sc_edge(a_tab, b_tab, e_term, didx)

    out = pl.pallas_call(
        _final_body,
        grid=(ngrid,),
        in_specs=[
            pl.BlockSpec((nblk, d), lambda i: (i, 0)),
            pl.BlockSpec((_NC, nblk, d), lambda i: (0, i, 0)),
            pl.BlockSpec((nblk, 1), lambda i: (i, 0)),
            pl.BlockSpec((d, d), lambda i: (0, 0)),
            pl.BlockSpec((d, d), lambda i: (0, 0)),
            pl.BlockSpec((n_graphs, d_u), lambda i: (0, 0)),
            pl.BlockSpec((d_u, d), lambda i: (0, 0)),
            pl.BlockSpec((1, d), lambda i: (0, 0)),
        ],
        out_specs=pl.BlockSpec((nblk, d), lambda i: (i, 0)),
        out_shape=jax.ShapeDtypeStruct((n_nodes, d), jnp.float32),
    )(x, agg_parts, bt2, W2x, W2m, u, W2u, b2r)
    return out


# speedup vs baseline: 2.1452x; 2.1452x over previous
import functools

import jax
import jax.numpy as jnp
from jax import lax
from jax.experimental import pallas as pl
from jax.experimental.pallas import tpu as pltpu
from jax.experimental.pallas import tpu_sc as plsc

_NC = 2
_NS = 16
_NW = _NC * _NS
_L = 16
_K = 64


def _node_tables_body(x_ref, bt_ref, w1d_ref, w1s_ref, u_ref, w1u_ref,
                      b1_ref, a_ref, b_ref):
    n_graphs = u_ref.shape[0]
    cu = jnp.dot(u_ref[...], w1u_ref[...],
                 preferred_element_type=jnp.float32) + b1_ref[...]
    oh = (bt_ref[...] == lax.broadcasted_iota(jnp.int32, (1, n_graphs), 1)
          ).astype(jnp.float32)
    xb = x_ref[...]
    a_ref[...] = (jnp.dot(xb, w1d_ref[...], preferred_element_type=jnp.float32)
                  + jnp.dot(oh, cu, preferred_element_type=jnp.float32))
    b_ref[...] = jnp.dot(xb, w1s_ref[...], preferred_element_type=jnp.float32)


def _make_edge_term_body(n_edges, eblk):
    def _edge_term_body(ea_ref, w1e_ref, e_ref):
        i = pl.program_id(0)
        row = i * eblk + lax.broadcasted_iota(jnp.int32, (eblk, 1), 0)
        pad = jnp.where(row >= n_edges, jnp.float32(-1e30), jnp.float32(0.0))
        e_ref[...] = jnp.dot(ea_ref[...], w1e_ref[...],
                             preferred_element_type=jnp.float32) + pad
    return _edge_term_body


def _final_body(x_ref, agg_ref, bt_ref, w2x_ref, w2m_ref, u_ref, w2u_ref,
                b2_ref, o_ref):
    n_graphs = u_ref.shape[0]
    cu = jnp.dot(u_ref[...], w2u_ref[...],
                 preferred_element_type=jnp.float32) + b2_ref[...]
    oh = (bt_ref[...] == lax.broadcasted_iota(jnp.int32, (1, n_graphs), 1)
          ).astype(jnp.float32)
    agg = agg_ref[0] + agg_ref[1]
    acc = jnp.dot(x_ref[...], w2x_ref[...], preferred_element_type=jnp.float32)
    acc = acc + jnp.dot(agg, w2m_ref[...], preferred_element_type=jnp.float32)
    acc = acc + jnp.dot(oh, cu, preferred_element_type=jnp.float32)
    o_ref[...] = jnp.maximum(acc, 0.0)


def _make_sc_edge_kernel(n_nodes, n_chunks, d):
    assert n_chunks % _NW == 0
    n_iter = n_chunks // _NW
    assert n_iter % 2 == 0
    rpt = (n_nodes // _NS) // 8 * 8
    tail = n_nodes - _NS * rpt
    wchunks = [(o, _K) for o in range(0, rpt - rpt % _K, _K)]
    if rpt % _K:
        wchunks.append((rpt - rpt % _K, rpt % _K))
    nj = d // _L

    mesh = plsc.VectorSubcoreMesh(core_axis_name="c", subcore_axis_name="s",
                                  num_cores=_NC, num_subcores=_NS)

    @functools.partial(
        pl.kernel,
        out_type=jax.ShapeDtypeStruct((_NC, n_nodes, d), jnp.float32),
        mesh=mesh,
        scratch_types=(
            [pltpu.VMEM((2, _K), jnp.int32)] * 2
            + [pltpu.VMEM((_K, d), jnp.float32)] * 6
            + [pltpu.VMEM_SHARED((n_nodes, d), jnp.float32)]
            + [pltpu.SemaphoreType.DMA] * 6
        ),
    )
    def sc_edge_kernel(a_hbm, b_hbm, e_hbm, didx_hbm, out_hbm, *scr):
        di = scr[0:2]
        ba = scr[2:4]
        bb = scr[4:6]
        be = scr[6:8]
        aggs = scr[8]
        sa = scr[9:11]
        sb = scr[11:13]
        se = scr[13:15]
        c = lax.axis_index("c")
        s = lax.axis_index("s")
        wid = s * _NC + c

        def gather_start(i, p):
            base = (i * _NW + wid) * _K
            pltpu.sync_copy(didx_hbm.at[i * _NW + wid], di[p])
            pltpu.make_async_copy(e_hbm.at[pl.ds(base, _K), :], be[p],
                                  se[p]).start()
            pltpu.make_async_copy(b_hbm.at[di[p].at[1]], bb[p], sb[p]).start()
            pltpu.make_async_copy(a_hbm.at[di[p].at[0]], ba[p], sa[p]).start()

        def gather_wait(i, p):
            base = (i * _NW + wid) * _K
            pltpu.make_async_copy(e_hbm.at[pl.ds(base, _K), :], be[p],
                                  se[p]).wait()
            pltpu.make_async_copy(b_hbm.at[di[p].at[1]], bb[p], sb[p]).wait()
            pltpu.make_async_copy(a_hbm.at[di[p].at[0]], ba[p], sa[p]).wait()

        gather_start(0, 0)

        def zero_row(r, carry):
            for j in range(nj):
                bb[1][r, pl.ds(j * _L, _L)] = jnp.zeros((_L,), jnp.float32)
            return carry
        lax.fori_loop(0, _K, zero_row, 0)
        for off, sz in wchunks:
            pltpu.sync_copy(bb[1].at[pl.ds(0, sz), :],
                            aggs.at[pl.ds(s * rpt + off, sz), :])
        if tail:
            @pl.when(s == 0)
            def _zero_tail():
                pltpu.sync_copy(bb[1].at[pl.ds(0, tail), :],
                                aggs.at[pl.ds(_NS * rpt, tail), :])
        plsc.subcore_barrier()

        def sub_iter(i, p):
            @pl.when(i + 1 < n_iter)
            def _():
                gather_start(i + 1, 1 - p)
            gather_wait(i, p)

            def msg_rows(r2, carry2):
                r = r2 * 2
                for rr in range(2):
                    for j in range(nj):
                        sl = pl.ds(j * _L, _L)
                        v = (ba[p][r + rr, sl] + bb[p][r + rr, sl]
                             + be[p][r + rr, sl])
                        ba[p][r + rr, sl] = jnp.maximum(v, 0.0)
                return carry2
            lax.fori_loop(0, _K // 2, msg_rows, 0)

            pltpu.sync_copy(ba[p], aggs.at[di[p].at[0]], add=True)

        def loop_body(i2, carry):
            sub_iter(i2 * 2, 0)
            sub_iter(i2 * 2 + 1, 1)
            return carry
        lax.fori_loop(0, n_iter // 2, loop_body, 0)
        plsc.subcore_barrier()

        for off, sz in wchunks:
            r0 = s * rpt + off
            pltpu.sync_copy(aggs.at[pl.ds(r0, sz), :],
                            bb[1].at[pl.ds(0, sz), :])
            pltpu.sync_copy(bb[1].at[pl.ds(0, sz), :],
                            out_hbm.at[c, pl.ds(r0, sz), :])
        if tail:
            @pl.when(s == 0)
            def _write_tail():
                r0 = _NS * rpt
                pltpu.sync_copy(aggs.at[pl.ds(r0, tail), :],
                                bb[1].at[pl.ds(0, tail), :])
                pltpu.sync_copy(bb[1].at[pl.ds(0, tail), :],
                                out_hbm.at[c, pl.ds(r0, tail), :])

    return sc_edge_kernel


def kernel(x, edge_index, edge_attr, u, batch, W1, b1, W2, b2):
    n_nodes, d = x.shape
    n_edges, d_edge = edge_attr.shape
    n_graphs, d_u = u.shape
    assert n_nodes % _NS == 0

    grp = _NW * _K * 2
    n_pad = -(-n_edges // grp) * grp
    n_chunks = n_pad // _K

    dst = edge_index[1].astype(jnp.int32)
    src = edge_index[0].astype(jnp.int32)
    pad1 = jnp.zeros((n_pad - n_edges,), jnp.int32)
    didx = jnp.stack([
        jnp.concatenate([dst, pad1]).reshape(n_chunks, _K),
        jnp.concatenate([src, pad1]).reshape(n_chunks, _K)], axis=1)
    ea_pad = jnp.concatenate(
        [edge_attr, jnp.zeros((n_pad - n_edges, d_edge), jnp.float32)], axis=0)
    bt2 = batch.astype(jnp.int32).reshape(n_nodes, 1)

    W1d = W1[0:d]
    W1s = W1[d:2 * d]
    W1e = W1[2 * d:2 * d + d_edge]
    W1u = W1[2 * d + d_edge:]
    b1r = b1.reshape(1, d)
    W2x = W2[0:d]
    W2m = W2[d:2 * d]
    W2u = W2[2 * d:]
    b2r = b2.reshape(1, d)

    nblk = 1000
    ngrid = n_nodes // nblk
    a_tab, b_tab = pl.pallas_call(
        _node_tables_body,
        grid=(ngrid,),
        in_specs=[
            pl.BlockSpec((nblk, d), lambda i: (i, 0)),
            pl.BlockSpec((nblk, 1), lambda i: (i, 0)),
            pl.BlockSpec((d, d), lambda i: (0, 0)),
            pl.BlockSpec((d, d), lambda i: (0, 0)),
            pl.BlockSpec((n_graphs, d_u), lambda i: (0, 0)),
            pl.BlockSpec((d_u, d), lambda i: (0, 0)),
            pl.BlockSpec((1, d), lambda i: (0, 0)),
        ],
        out_specs=[
            pl.BlockSpec((nblk, d), lambda i: (i, 0)),
            pl.BlockSpec((nblk, d), lambda i: (i, 0)),
        ],
        out_shape=[
            jax.ShapeDtypeStruct((n_nodes, d), jnp.float32),
            jax.ShapeDtypeStruct((n_nodes, d), jnp.float32),
        ],
    )(x, bt2, W1d, W1s, u, W1u, b1r)

    eblk = 4096
    assert n_pad % eblk == 0
    egrid = n_pad // eblk
    e_term = pl.pallas_call(
        _make_edge_term_body(n_edges, eblk),
        grid=(egrid,),
        in_specs=[
            pl.BlockSpec((eblk, d_edge), lambda i: (i, 0)),
            pl.BlockSpec((d_edge, d), lambda i: (0, 0)),
        ],
        out_specs=pl.BlockSpec((eblk, d), lambda i: (i, 0)),
        out_shape=jax.ShapeDtypeStruct((n_pad, d), jnp.float32),
    )(ea_pad, W1e)

    sc_edge = _make_sc_edge_kernel(n_nodes, n_chunks, d)
    agg_parts = sc_edge(a_tab, b_tab, e_term, didx)

    out = pl.pallas_call(
        _final_body,
        grid=(ngrid,),
        in_specs=[
            pl.BlockSpec((nblk, d), lambda i: (i, 0)),
            pl.BlockSpec((_NC, nblk, d), lambda i: (0, i, 0)),
            pl.BlockSpec((nblk, 1), lambda i: (i, 0)),
            pl.BlockSpec((d, d), lambda i: (0, 0)),
            pl.BlockSpec((d, d), lambda i: (0, 0)),
            pl.BlockSpec((n_graphs, d_u), lambda i: (0, 0)),
            pl.BlockSpec((d_u, d), lambda i: (0, 0)),
            pl.BlockSpec((1, d), lambda i: (0, 0)),
        ],
        out_specs=pl.BlockSpec((nblk, d), lambda i: (i, 0)),
        out_shape=jax.ShapeDtypeStruct((n_nodes, d), jnp.float32),
    )(x, agg_parts, bt2, W2x, W2m, u, W2u, b2r)
    return out
